# baseline (speedup 1.0000x reference)
import functools

import jax
import jax.numpy as jnp
from jax import lax
from jax.experimental import pallas as pl
from jax.experimental.pallas import tpu as pltpu
from jax.experimental.pallas import tpu_sc as plsc

N = 10000
E = 320000
DIN = 128
HID = 128
NCLS = 64
NBR = 4
NC = 2
NS = 16
NW = NC * NS
EPT = E // NW
CH = 100
NCHUNK = EPT // CH
GW = 20
NWIN = NCHUNK // GW
NPAD = 10240
DPT = NPAD // NS
ZROWS = NPAD // NS

_MESH = plsc.VectorSubcoreMesh(core_axis_name="c", subcore_axis_name="s",
                               num_cores=NC, num_subcores=NS)


@functools.partial(
    pl.kernel,
    out_type=jax.ShapeDtypeStruct((NC, 2, NPAD), jnp.float32),
    mesh=_MESH,
    scratch_types=[
        pltpu.VMEM((NCHUNK, CH), jnp.int32),
        pltpu.VMEM((NCHUNK, CH), jnp.int32),
        pltpu.VMEM((CH,), jnp.float32),
        pltpu.VMEM_SHARED((NPAD,), jnp.float32),
        pltpu.VMEM_SHARED((NPAD,), jnp.float32),
    ],
)
def _deg_call(srcr, dstr, ones_h, zflat, out, idx_s, idx_d, ones_v, acc_o, acc_i):
    cid = lax.axis_index("c")
    sid = lax.axis_index("s")
    pltpu.sync_copy(srcr.at[cid, sid], idx_s)
    pltpu.sync_copy(dstr.at[cid, sid], idx_d)
    pltpu.sync_copy(ones_h, ones_v)
    pltpu.sync_copy(zflat, acc_o.at[pl.ds(sid * DPT, DPT)])
    pltpu.sync_copy(zflat, acc_i.at[pl.ds(sid * DPT, DPT)])
    plsc.subcore_barrier()

    @pl.loop(0, NCHUNK)
    def _chunk(j):
        pltpu.sync_copy(ones_v, acc_o.at[idx_s.at[j]], add=True)
        pltpu.sync_copy(ones_v, acc_i.at[idx_d.at[j]], add=True)

    plsc.subcore_barrier()
    pltpu.sync_copy(acc_o.at[pl.ds(sid * DPT, DPT)],
                    out.at[cid, 0, pl.ds(sid * DPT, DPT)])
    pltpu.sync_copy(acc_i.at[pl.ds(sid * DPT, DPT)],
                    out.at[cid, 1, pl.ds(sid * DPT, DPT)])


def _make_prop(nbr):
    @functools.partial(
        pl.kernel,
        out_type=jax.ShapeDtypeStruct((NC, nbr, NPAD, DIN), jnp.float32),
        mesh=_MESH,
        scratch_types=[
            pltpu.VMEM((NCHUNK, CH), jnp.int32),
            pltpu.VMEM((GW, CH), jnp.int32),
            pltpu.VMEM((GW, CH), jnp.int32),
            pltpu.VMEM((CH, DIN), jnp.float32),
            pltpu.VMEM((CH, DIN), jnp.float32),
            pltpu.VMEM_SHARED((NPAD, DIN), jnp.float32),
            pltpu.SemaphoreType.DMA,
            pltpu.SemaphoreType.DMA,
            pltpu.SemaphoreType.DMA,
            pltpu.SemaphoreType.DMA,
            pltpu.SemaphoreType.DMA,
            pltpu.SemaphoreType.DMA,
        ],
    )
    def _prop(*refs):
        hs = refs[:nbr]
        (srcr, dstr5, z2d, out, idx_s, idx_d0, idx_d1, rows_a, rows_b, acc,
         sga, sgb, ssa, ssb, sd0, sd1) = refs[nbr:]
        cid = lax.axis_index("c")
        sid = lax.axis_index("s")
        idx_d = (idx_d0, idx_d1)
        sem_d = (sd0, sd1)
        rows = (rows_a, rows_b)
        sg = (sga, sgb)
        ss = (ssa, ssb)
        pltpu.sync_copy(srcr.at[cid, sid], idx_s)

        def wait_g(h, j, p):
            pltpu.make_async_copy(h.at[idx_s.at[j]], rows[p], sg[p]).wait()

        def wait_s(dcur, p):
            pltpu.make_async_copy(rows[p], acc.at[dcur.at[0]], ss[p]).wait()

        for br in range(nbr):
            pltpu.sync_copy(z2d, acc.at[pl.ds(sid * ZROWS, ZROWS)])
            plsc.subcore_barrier()
            h = hs[br]
            pltpu.async_copy(dstr5.at[cid, sid, 0], idx_d0, sd0)
            pltpu.async_copy(h.at[idx_s.at[0]], rows_a, sga)
            for w in range(NWIN):
                dcur = idx_d[w % 2]
                pltpu.make_async_copy(dstr5.at[cid, sid, 0],
                                      dcur, sem_d[w % 2]).wait()
                j0 = w * GW
                wait_g(h, j0, 0)
                pltpu.async_copy(rows_a, acc.at[dcur.at[0]], ssa, add=True)
                if w > 0:
                    wait_s(dcur, 1)
                if w + 1 < NWIN:
                    pltpu.async_copy(dstr5.at[cid, sid, w + 1],
                                     idx_d[(w + 1) % 2], sem_d[(w + 1) % 2])
                pltpu.async_copy(h.at[idx_s.at[j0 + 1]], rows_b, sgb)

                @pl.loop(0, (GW - 2) // 2)
                def _pair(m, _h=h, _d=dcur, _w=w):
                    jb = _w * GW + 1 + 2 * m
                    wait_g(_h, jb, 1)
                    pltpu.async_copy(rows_b, acc.at[_d.at[jb - _w * GW]],
                                     ssb, add=True)
                    wait_s(_d, 0)
                    pltpu.async_copy(_h.at[idx_s.at[jb + 1]], rows_a, sga)
                    ja = jb + 1
                    wait_g(_h, ja, 0)
                    pltpu.async_copy(rows_a, acc.at[_d.at[ja - _w * GW]],
                                     ssa, add=True)
                    wait_s(_d, 1)
                    pltpu.async_copy(_h.at[idx_s.at[ja + 1]], rows_b, sgb)

                jl = w * GW + GW - 1
                wait_g(h, jl, 1)
                pltpu.async_copy(rows_b, acc.at[dcur.at[GW - 1]], ssb, add=True)
                wait_s(dcur, 0)
                if w + 1 < NWIN:
                    pltpu.async_copy(h.at[idx_s.at[jl + 1]], rows_a, sga)

            wait_s(idx_d[(NWIN - 1) % 2], 1)
            plsc.subcore_barrier()
            pltpu.sync_copy(acc.at[pl.ds(sid * ZROWS, ZROWS)],
                            out.at[cid, br, pl.ds(sid * ZROWS, ZROWS)])

    return _prop


_prop1 = _make_prop(1)
_prop4 = _make_prop(NBR)


def _norm_body(degp_ref, x_ref, nrm_ref, xs_ref):
    d = jnp.sum(degp_ref[...], axis=0)
    nrm = lax.rsqrt(jnp.maximum(d, 1.0))
    nrm_ref[...] = nrm
    ns = nrm[0, :N]
    xs_ref[...] = x_ref[...] * ns[:, None]


_norm_call = pl.pallas_call(
    _norm_body,
    out_shape=(jax.ShapeDtypeStruct((2, NPAD), jnp.float32),
               jax.ShapeDtypeStruct((N, DIN), jnp.float32)),
)


def _make_mm(p_nbr, out_dim, with_relu):
    RB = 1024

    def body(*refs):
        if with_relu:
            pp_ref, w_ref, b_ref, nrm_ref, f_ref, h_ref = refs
        else:
            pp_ref, w_ref, b_ref, nrm_ref, f_ref = refs
        p = pp_ref[0, 0] + pp_ref[1, 0]
        nd = nrm_ref[1, 0, 0]
        f = jnp.dot(p * nd[:, None], w_ref[0],
                    preferred_element_type=jnp.float32) + b_ref[0, 0][None, :]
        f_ref[0] = f
        if with_relu:
            ns = nrm_ref[0, 0, 0]
            h_ref[0] = jnp.maximum(f, 0.0) * ns[:, None]

    out_shape = [jax.ShapeDtypeStruct((NBR, NPAD, out_dim), jnp.float32)]
    out_specs = [pl.BlockSpec((1, RB, out_dim), lambda br, r: (br, r, 0))]
    if with_relu:
        out_shape.append(jax.ShapeDtypeStruct((NBR, NPAD, out_dim), jnp.float32))
        out_specs.append(pl.BlockSpec((1, RB, out_dim), lambda br, r: (br, r, 0)))

    return pl.pallas_call(
        body,
        grid=(NBR, NPAD // RB),
        in_specs=[
            pl.BlockSpec((NC, 1, RB, DIN),
                         (lambda br, r: (0, br, r, 0)) if p_nbr > 1
                         else (lambda br, r: (0, 0, r, 0))),
            pl.BlockSpec((1, DIN, out_dim), lambda br, r: (br, 0, 0)),
            pl.BlockSpec((1, 1, out_dim), lambda br, r: (br, 0, 0)),
            pl.BlockSpec((2, 1, 1, RB), lambda br, r: (0, r, 0, 0)),
        ],
        out_specs=out_specs,
        out_shape=tuple(out_shape),
    )


_mm0 = _make_mm(1, HID, True)
_mm1 = _make_mm(NBR, HID, True)
_mm2 = _make_mm(NBR, NCLS, False)


def kernel(x, edge_index, W0, b0, W1, b1, W2, b2):
    x = x.astype(jnp.float32)
    ei = edge_index.astype(jnp.int32).reshape(2, NC, NS, NCHUNK, CH)
    srcr, dstr = ei[0], ei[1]
    ones_h = jnp.ones((CH,), jnp.float32)
    zflat = jnp.zeros((DPT,), jnp.float32)
    z2d = jnp.zeros((ZROWS, DIN), jnp.float32)

    dstr5 = dstr.reshape(NC, NS, NWIN, GW, CH)
    degp = _deg_call(srcr, dstr, ones_h, zflat)
    norms, xs = _norm_call(degp, x)
    norms34 = norms.reshape(2, NPAD // 1024, 1, 1024)
    b0r, b1r, b2r = (b[:, None] for b in (b0, b1, b2))

    p0 = _prop1(xs, srcr, dstr5, z2d)
    feats0, h1 = _mm0(p0, W0, b0r, norms34)
    q = _prop4(h1[0], h1[1], h1[2], h1[3], srcr, dstr5, z2d)
    feats1, h2 = _mm1(q, W1, b1r, norms34)
    r = _prop4(h2[0], h2[1], h2[2], h2[3], srcr, dstr5, z2d)
    output, = _mm2(r, W2, b2r, norms34)

    featss = tuple((feats0[i, :N], feats1[i, :N]) for i in range(NBR))
    return (output[:, :N], featss)


# speedup vs baseline: 1.0920x; 1.0920x over previous
import functools

import jax
import jax.numpy as jnp
from jax import lax
from jax.experimental import pallas as pl
from jax.experimental.pallas import tpu as pltpu
from jax.experimental.pallas import tpu_sc as plsc

N = 10000
E = 320000
DIN = 128
HID = 128
NCLS = 64
NBR = 4
NC = 2
NS = 16
NW = NC * NS
EPT = E // NW
CH = 125
NCHUNK = EPT // CH
GW = 16
NWIN = NCHUNK // GW
NPAD = 10240
DPT = NPAD // NS
ZROWS = NPAD // NS

_MESH = plsc.VectorSubcoreMesh(core_axis_name="c", subcore_axis_name="s",
                               num_cores=NC, num_subcores=NS)


@functools.partial(
    pl.kernel,
    out_type=jax.ShapeDtypeStruct((NC, 2, NPAD), jnp.float32),
    mesh=_MESH,
    scratch_types=[
        pltpu.VMEM((NCHUNK, CH), jnp.int32),
        pltpu.VMEM((NCHUNK, CH), jnp.int32),
        pltpu.VMEM((CH,), jnp.float32),
        pltpu.VMEM_SHARED((NPAD,), jnp.float32),
        pltpu.VMEM_SHARED((NPAD,), jnp.float32),
    ],
)
def _deg_call(srcr, dstr, ones_h, zflat, out, idx_s, idx_d, ones_v, acc_o, acc_i):
    cid = lax.axis_index("c")
    sid = lax.axis_index("s")
    pltpu.sync_copy(srcr.at[cid, sid], idx_s)
    pltpu.sync_copy(dstr.at[cid, sid], idx_d)
    pltpu.sync_copy(ones_h, ones_v)
    pltpu.sync_copy(zflat, acc_o.at[pl.ds(sid * DPT, DPT)])
    pltpu.sync_copy(zflat, acc_i.at[pl.ds(sid * DPT, DPT)])
    plsc.subcore_barrier()

    @pl.loop(0, NCHUNK)
    def _chunk(j):
        pltpu.sync_copy(ones_v, acc_o.at[idx_s.at[j]], add=True)
        pltpu.sync_copy(ones_v, acc_i.at[idx_d.at[j]], add=True)

    plsc.subcore_barrier()
    pltpu.sync_copy(acc_o.at[pl.ds(sid * DPT, DPT)],
                    out.at[cid, 0, pl.ds(sid * DPT, DPT)])
    pltpu.sync_copy(acc_i.at[pl.ds(sid * DPT, DPT)],
                    out.at[cid, 1, pl.ds(sid * DPT, DPT)])


def _make_prop(nbr):
    @functools.partial(
        pl.kernel,
        out_type=jax.ShapeDtypeStruct((NC, nbr, NPAD, DIN), jnp.float32),
        mesh=_MESH,
        scratch_types=[
            pltpu.VMEM((NCHUNK, CH), jnp.int32),
            pltpu.VMEM((GW, CH), jnp.int32),
            pltpu.VMEM((GW, CH), jnp.int32),
            pltpu.VMEM((CH, DIN), jnp.float32),
            pltpu.VMEM((CH, DIN), jnp.float32),
            pltpu.VMEM_SHARED((NPAD, DIN), jnp.float32),
            pltpu.SemaphoreType.DMA,
            pltpu.SemaphoreType.DMA,
            pltpu.SemaphoreType.DMA,
            pltpu.SemaphoreType.DMA,
        ],
    )
    def _prop(*refs):
        hs = refs[:nbr]
        (srcr, dstr, z2d, out, idx_s, idx_d0, idx_d1, rows_a, rows_b, acc,
         sem_a, sem_b, sem_d0, sem_d1) = refs[nbr:]
        cid = lax.axis_index("c")
        sid = lax.axis_index("s")
        idx_d = (idx_d0, idx_d1)
        sem_d = (sem_d0, sem_d1)
        pltpu.sync_copy(srcr.at[cid, sid], idx_s)
        for br in range(nbr):
            pltpu.sync_copy(z2d, acc.at[pl.ds(sid * ZROWS, ZROWS)])
            plsc.subcore_barrier()
            h = hs[br]
            pltpu.async_copy(dstr.at[cid, sid, pl.ds(0, GW)], idx_d0, sem_d0)
            pltpu.async_copy(h.at[idx_s.at[0]], rows_a, sem_a)
            for w in range(NWIN):
                dcur, dnxt = idx_d[w % 2], idx_d[(w + 1) % 2]
                scur, snxt = sem_d[w % 2], sem_d[(w + 1) % 2]
                pltpu.make_async_copy(dstr.at[cid, sid, pl.ds(0, GW)],
                                      dcur, scur).wait()
                if w + 1 < NWIN:
                    pltpu.async_copy(dstr.at[cid, sid, pl.ds((w + 1) * GW, GW)],
                                     dnxt, snxt)

                @pl.loop(0, GW // 2)
                def _pair(k, _h=h, _w=w, _d=dcur):
                    j = _w * GW + 2 * k
                    pltpu.make_async_copy(_h.at[idx_s.at[j]], rows_a, sem_a).wait()
                    pltpu.async_copy(_h.at[idx_s.at[j + 1]], rows_b, sem_b)
                    pltpu.sync_copy(rows_a, acc.at[_d.at[2 * k]], add=True)
                    pltpu.make_async_copy(_h.at[idx_s.at[j + 1]], rows_b, sem_b).wait()

                    @pl.when(j + 2 < NCHUNK)
                    def _():
                        pltpu.async_copy(_h.at[idx_s.at[j + 2]], rows_a, sem_a)

                    pltpu.sync_copy(rows_b, acc.at[_d.at[2 * k + 1]], add=True)

            plsc.subcore_barrier()
            pltpu.sync_copy(acc.at[pl.ds(sid * ZROWS, ZROWS)],
                            out.at[cid, br, pl.ds(sid * ZROWS, ZROWS)])

    return _prop


_prop1 = _make_prop(1)
_prop4 = _make_prop(NBR)


def _norm_body(degp_ref, x_ref, nrm_ref, xs_ref):
    d = jnp.sum(degp_ref[...], axis=0)
    nrm = lax.rsqrt(jnp.maximum(d, 1.0))
    nrm_ref[...] = nrm
    ns = nrm[0, :N]
    xs_ref[...] = x_ref[...] * ns[:, None]


_norm_call = pl.pallas_call(
    _norm_body,
    out_shape=(jax.ShapeDtypeStruct((2, NPAD), jnp.float32),
               jax.ShapeDtypeStruct((N, DIN), jnp.float32)),
)


def _make_mm(p_nbr, out_dim, with_relu):
    RB = 1000

    def body(*refs):
        if with_relu:
            pp_ref, w_ref, b_ref, nrm_ref, f_ref, h_ref = refs
        else:
            pp_ref, w_ref, b_ref, nrm_ref, f_ref = refs
        p = pp_ref[0, 0] + pp_ref[1, 0]
        nd = nrm_ref[1, 0, 0]
        f = jnp.dot(p * nd[:, None], w_ref[0],
                    preferred_element_type=jnp.float32) + b_ref[0, 0][None, :]
        f_ref[0] = f
        if with_relu:
            ns = nrm_ref[0, 0, 0]
            h_ref[0] = jnp.maximum(f, 0.0) * ns[:, None]

    out_shape = [jax.ShapeDtypeStruct((NBR, N, out_dim), jnp.float32)]
    out_specs = [pl.BlockSpec((1, RB, out_dim), lambda br, r: (br, r, 0))]
    if with_relu:
        out_shape.append(jax.ShapeDtypeStruct((NBR, N, out_dim), jnp.float32))
        out_specs.append(pl.BlockSpec((1, RB, out_dim), lambda br, r: (br, r, 0)))

    return pl.pallas_call(
        body,
        grid=(NBR, N // RB),
        in_specs=[
            pl.BlockSpec((NC, 1, RB, DIN),
                         (lambda br, r: (0, br, r, 0)) if p_nbr > 1
                         else (lambda br, r: (0, 0, r, 0))),
            pl.BlockSpec((1, DIN, out_dim), lambda br, r: (br, 0, 0)),
            pl.BlockSpec((1, 1, out_dim), lambda br, r: (br, 0, 0)),
            pl.BlockSpec((2, 1, 1, RB), lambda br, r: (0, r, 0, 0)),
        ],
        out_specs=out_specs,
        out_shape=tuple(out_shape),
    )


_mm0 = _make_mm(1, HID, True)
_mm1 = _make_mm(NBR, HID, True)
_mm2 = _make_mm(NBR, NCLS, False)


def kernel(x, edge_index, W0, b0, W1, b1, W2, b2):
    x = x.astype(jnp.float32)
    ei = edge_index.astype(jnp.int32).reshape(2, NC, NS, NCHUNK, CH)
    srcr, dstr = ei[0], ei[1]
    ones_h = jnp.ones((CH,), jnp.float32)
    zflat = jnp.zeros((DPT,), jnp.float32)
    z2d = jnp.zeros((ZROWS, DIN), jnp.float32)

    degp = _deg_call(srcr, dstr, ones_h, zflat)
    norms, xs = _norm_call(degp, x)
    norms34 = norms[:, :N].reshape(2, N // 1000, 1, 1000)
    b0r, b1r, b2r = (b[:, None] for b in (b0, b1, b2))

    p0 = _prop1(xs, srcr, dstr, z2d)
    feats0, h1 = _mm0(p0, W0, b0r, norms34)
    q = _prop4(h1[0], h1[1], h1[2], h1[3], srcr, dstr, z2d)
    feats1, h2 = _mm1(q, W1, b1r, norms34)
    r = _prop4(h2[0], h2[1], h2[2], h2[3], srcr, dstr, z2d)
    output, = _mm2(r, W2, b2r, norms34)

    featss = tuple((feats0[i], feats1[i]) for i in range(NBR))
    return (output, featss)
